# Initial kernel scaffold; baseline (speedup 1.0000x reference)
import jax
import jax.numpy as jnp
from jax.experimental import pallas as pl


def kernel(x, W, b):
    raise NotImplementedError("write your pallas kernel here")


# speedup vs baseline: 14.4351x; 14.4351x over previous
import functools

import jax
import jax.numpy as jnp
from jax.experimental import pallas as pl
from jax.experimental.pallas import tpu as pltpu

IN_DIMS = 1024
SEQ_LEN = 2048
TOP_K = 64
BLK = 256
N_BISECT = 18


def _fused_body(xb_ref, xf_ref, w_ref, b_ref, out_ref):
    xb = xb_ref[0]
    xf = xf_ref[0]
    xff = xf.astype(jnp.float32)
    xbf = xb.astype(jnp.float32)
    rn_b = jax.lax.rsqrt(jnp.maximum(
        jnp.sum(xbf * xbf, axis=1, keepdims=True), 1e-24))
    rn_f = jax.lax.rsqrt(jnp.maximum(
        jnp.sum(xff * xff, axis=1, keepdims=True), 1e-24))
    scores = jax.lax.dot_general(
        xb, xf, (((1,), (1,)), ((), ())),
        preferred_element_type=jnp.float32)
    scores = scores * rn_b * rn_f.T

    lo0 = jnp.full((BLK, 1), -1.01, jnp.float32)
    hi0 = jnp.full((BLK, 1), 1.01, jnp.float32)

    def body(_, carry):
        lo, hi = carry
        mid = 0.5 * (lo + hi)
        cnt = jnp.sum((scores >= mid).astype(jnp.float32), axis=1,
                      keepdims=True)
        ge = cnt >= TOP_K
        return jnp.where(ge, mid, lo), jnp.where(ge, hi, mid)

    lo, _ = jax.lax.fori_loop(0, N_BISECT, body, (lo0, hi0))

    m = scores >= lo
    e = jnp.exp(scores)
    w = jnp.where(m, e - 1.0, 0.0)
    mf = m.astype(jnp.float32)
    cnt = jnp.sum(mf, axis=1, keepdims=True)
    sumexp = jnp.sum(jnp.where(m, e, 0.0), axis=1, keepdims=True)
    denom = sumexp + (SEQ_LEN - cnt)

    colsum = jnp.sum(xff, axis=0, keepdims=True)
    wx = jax.lax.dot_general(
        w.astype(jnp.bfloat16), xf, (((1,), (0,)), ((), ())),
        preferred_element_type=jnp.float32)
    g = (wx + colsum) / denom
    out = jax.lax.dot_general(
        g.astype(jnp.bfloat16), w_ref[...], (((1,), (1,)), ((), ())),
        preferred_element_type=jnp.float32)
    out_ref[0] = out + b_ref[...]


def kernel(x, W, b):
    B, S, D = x.shape
    nblk = S // BLK
    b2 = b.reshape(1, D)
    xb16 = x.astype(jnp.bfloat16)
    Wb16 = W.astype(jnp.bfloat16)
    out = pl.pallas_call(
        _fused_body,
        grid=(B, nblk),
        in_specs=[
            pl.BlockSpec((1, BLK, D), lambda bi, i: (bi, i, 0)),
            pl.BlockSpec((1, S, D), lambda bi, i: (bi, 0, 0)),
            pl.BlockSpec((D, D), lambda bi, i: (0, 0)),
            pl.BlockSpec((1, D), lambda bi, i: (0, 0)),
        ],
        out_specs=pl.BlockSpec((1, BLK, D), lambda bi, i: (bi, i, 0)),
        out_shape=jax.ShapeDtypeStruct((B, S, D), jnp.float32),
    )(xb16, xb16, Wb16, b2)
    return out
